# baseline (speedup 1.0000x reference)
import functools
import jax
import jax.numpy as jnp
from jax import lax
from jax.experimental import pallas as pl
from jax.experimental.pallas import tpu as pltpu
from jax.experimental.pallas import tpu_sc as plsc

B, J = 16, 8
ROWS = 3072
RPW = ROWS // 2
CR = 32
NCH = RPW // CR
NBUF = 3
NOBUF = 2

_mesh = plsc.VectorSubcoreMesh(core_axis_name="c", subcore_axis_name="s")


@functools.partial(
    pl.kernel,
    out_type=jax.ShapeDtypeStruct((B, ROWS, 128), jnp.float32),
    mesh=_mesh,
    scratch_types=[
        pltpu.VMEM((J, B), jnp.float32),
        pltpu.VMEM((B,), jnp.float32),
        pltpu.VMEM((J * B + B,), jnp.float32),
        pltpu.VMEM((NBUF, J, CR, 128), jnp.float32),
        pltpu.VMEM((NOBUF, CR, 128), jnp.float32),
        pltpu.SemaphoreType.DMA((NBUF,)),
        pltpu.SemaphoreType.DMA((NOBUF,)),
    ],
)
def _sc_kernel(x_hbm, maskT_hbm, alpha_hbm, out_hbm,
               mask_v, alpha_v, w_v, in_buf, out_buf, isem, osem):
    wid = lax.axis_index("s") * 2 + lax.axis_index("c")
    b = wid // 2
    h = wid % 2
    r0 = h * RPW

    pltpu.sync_copy(maskT_hbm, mask_v)
    pltpu.sync_copy(alpha_hbm, alpha_v)
    av = alpha_v[...]
    logits = [mask_v[j] * av for j in range(J)]
    mx = logits[0]
    for j in range(1, J):
        mx = jnp.maximum(mx, logits[j])
    es = [jnp.exp(l - mx) for l in logits]
    den = es[0]
    for j in range(1, J):
        den = den + es[j]
    for j in range(J):
        w_v[pl.ds(j * B, B)] = es[j] / den
    wv = [jnp.full((B,), w_v[pl.ds(j * B + b, B)][0]) for j in range(J)]

    def issue_in(c):
        slot = lax.rem(c, NBUF)
        pltpu.make_async_copy(
            x_hbm.at[b, :, pl.ds(r0 + c * CR, CR), :],
            in_buf.at[slot], isem.at[slot],
        ).start()

    def wait_in(c):
        slot = lax.rem(c, NBUF)
        pltpu.make_async_copy(
            x_hbm.at[b, :, pl.ds(r0 + c * CR, CR), :],
            in_buf.at[slot], isem.at[slot],
        ).wait()

    def issue_out(c):
        oslot = lax.rem(c, NOBUF)
        pltpu.make_async_copy(
            out_buf.at[oslot], out_hbm.at[b, pl.ds(r0 + c * CR, CR), :],
            osem.at[oslot],
        ).start()

    def wait_out_slot(oslot):
        pltpu.make_async_copy(
            out_buf.at[oslot], out_hbm.at[b, pl.ds(r0, CR), :],
            osem.at[oslot],
        ).wait()

    for p in range(NBUF - 1):
        issue_in(jnp.int32(p))

    def chunk(c, carry):
        slot = lax.rem(c, NBUF)
        oslot = lax.rem(c, NOBUF)

        @pl.when(c + NBUF - 1 < NCH)
        def _():
            issue_in(c + NBUF - 1)

        wait_in(c)

        @pl.when(c >= NOBUF)
        def _():
            wait_out_slot(oslot)

        def row(rr, carry2):
            for q in range(8):
                base = q * 16
                v = wv[0] * in_buf[slot, 0, rr, pl.ds(base, 16)]
                for j in range(1, J):
                    v = v + wv[j] * in_buf[slot, j, rr, pl.ds(base, 16)]
                out_buf[oslot, rr, pl.ds(base, 16)] = v
            return carry2

        lax.fori_loop(0, CR, row, 0)
        issue_out(c)
        return carry

    lax.fori_loop(0, NCH, chunk, 0)

    for k in range(NOBUF):
        wait_out_slot(jnp.int32(k))


def kernel(inps, mask, alpha):
    x = inps.reshape(B, J, ROWS, 128)
    maskT = mask.T
    alpha16 = jnp.full((B,), alpha, dtype=jnp.float32)
    out = _sc_kernel(x, maskT, alpha16)
    sampled = out.reshape(B, 96, 64, 64)
    logp = jnp.zeros((B,), jnp.float32)
    return (sampled, logp)
